# baseline (speedup 1.0000x reference)
import functools

import jax
import jax.numpy as jnp
from jax import lax
from jax.experimental import pallas as pl
from jax.experimental.pallas import tpu as pltpu
from jax.experimental.pallas import tpu_sc as plsc

R, C = 4096, 200
N = R * C
L = 16
NC, NS = 2, 16
NW = NC * NS
CPW = N // NW
TPW = CPW // L
ROWS_PW = R // NW
IGNORE = -1.0


def _count_body(x_ref, out_ref):
    w = pl.program_id(0)

    @pl.when(w == 0)
    def _():
        out_ref[0] = 0

    c = jnp.sum((x_ref[...] != IGNORE).astype(jnp.int32))
    out_ref[w + 1] = out_ref[w] + c


def _prefix(x):
    return pl.pallas_call(
        _count_body,
        grid=(NW,),
        in_specs=[pl.BlockSpec((ROWS_PW, C), lambda i: (i, 0))],
        out_specs=pl.BlockSpec(memory_space=pltpu.SMEM),
        out_shape=jax.ShapeDtypeStruct((48,), jnp.int32),
    )(x)


def _gather16(x, idx):
    return x.at[idx].get(mode="promise_in_bounds")


def _cumsum16(x, lane):
    s = x
    for k in (1, 2, 4, 8):
        sh = _gather16(s, jnp.maximum(lane - k, 0))
        s = s + jnp.where(lane >= k, sh, 0)
    return s


def _mult8(x):
    return pl.multiple_of(x, 8)


SBUF = CPW + 32


@functools.partial(
    pl.kernel,
    mesh=plsc.VectorSubcoreMesh(core_axis_name="c", subcore_axis_name="s"),
    out_type=[
        jax.ShapeDtypeStruct((N,), jnp.float32),
        jax.ShapeDtypeStruct((N,), jnp.int32),
        jax.ShapeDtypeStruct((N,), jnp.int32),
    ],
    scratch_types=[
        pltpu.VMEM((CPW,), jnp.float32),
        pltpu.VMEM((SBUF,), jnp.float32),
        pltpu.VMEM((SBUF,), jnp.int32),
        pltpu.VMEM((SBUF,), jnp.int32),
        pltpu.VMEM((48,), jnp.int32),
        pltpu.SemaphoreType.DMA,
    ],
    compiler_params=pltpu.CompilerParams(needs_layout_passes=False),
)
def _transform(x_hbm, pref_hbm, val_hbm, rows_hbm, cols_hbm,
               chunk, vbuf, rbuf, cbuf, pref, sem):
    wid = lax.axis_index("s") * NC + lax.axis_index("c")
    base_elem = _mult8(wid * CPW)

    pltpu.sync_copy(pref_hbm, pref)
    lane = lax.iota(jnp.int32, L)
    K = pref[2 * L : 3 * L][0]

    @pl.when(K == N)
    def _fast():
        pltpu.sync_copy(x_hbm.at[pl.ds(base_elem, CPW)],
                        val_hbm.at[pl.ds(base_elem, CPW)])

        def body(t, carry):
            rows, cols = carry
            j = t * L
            rbuf[pl.ds(j, L)] = rows
            cbuf[pl.ds(j, L)] = cols
            cols2 = cols + L
            wrap = cols2 >= C
            return rows + wrap.astype(jnp.int32), jnp.where(wrap, cols2 - C, cols2)

        lax.fori_loop(0, TPW, body,
                      (jnp.broadcast_to(wid * ROWS_PW, (L,)), lane))
        pltpu.sync_copy(rbuf.at[pl.ds(0, CPW)],
                        rows_hbm.at[pl.ds(base_elem, CPW)])
        pltpu.sync_copy(cbuf.at[pl.ds(0, CPW)],
                        cols_hbm.at[pl.ds(base_elem, CPW)])

    @pl.when(jnp.logical_and(K != N, wid == 0))
    def _slow():
        pltpu.sync_copy(x_hbm.at[pl.ds(0, L)], chunk.at[pl.ds(0, L)])
        padv = jnp.broadcast_to(chunk[0:L][0], (L,))
        zerov = jnp.zeros((L,), jnp.int32)

        def flush(fill, flushed):
            nbig = fill >> 10

            def fbig(i, _):
                src = _mult8(i * 1024)
                dst = _mult8(flushed + i * 1024)
                pltpu.sync_copy(vbuf.at[pl.ds(src, 1024)],
                                val_hbm.at[pl.ds(dst, 1024)])
                pltpu.sync_copy(rbuf.at[pl.ds(src, 1024)],
                                rows_hbm.at[pl.ds(dst, 1024)])
                pltpu.sync_copy(cbuf.at[pl.ds(src, 1024)],
                                cols_hbm.at[pl.ds(dst, 1024)])
                return 0

            lax.fori_loop(0, nbig, fbig, 0)
            base8 = nbig << 10
            nsm = (fill - base8) >> 3

            def fsm(i, _):
                src = _mult8(base8 + i * 8)
                dst = _mult8(flushed + base8 + i * 8)
                pltpu.sync_copy(vbuf.at[pl.ds(src, 8)],
                                val_hbm.at[pl.ds(dst, 8)])
                pltpu.sync_copy(rbuf.at[pl.ds(src, 8)],
                                rows_hbm.at[pl.ds(dst, 8)])
                pltpu.sync_copy(cbuf.at[pl.ds(src, 8)],
                                cols_hbm.at[pl.ds(dst, 8)])
                return 0

            lax.fori_loop(0, nsm, fsm, 0)
            done = _mult8((fill >> 3) << 3)
            tv = vbuf[pl.ds(done, L)]
            tr = rbuf[pl.ds(done, L)]
            tc = cbuf[pl.ds(done, L)]
            vbuf[0:L] = tv
            rbuf[0:L] = tr
            cbuf[0:L] = tc
            return fill - done, flushed + done

        def chunk_body(ch, carry):
            fill, flushed = carry
            pltpu.sync_copy(x_hbm.at[pl.ds(_mult8(ch * CPW), CPW)], chunk)
            rows0 = jnp.broadcast_to(ch * ROWS_PW, (L,))

            def body(t, c2):
                fill, rows, cols = c2
                v = chunk[pl.ds(t * L, L)]
                m = v != IGNORE
                mi = m.astype(jnp.int32)
                incl = _cumsum16(mi, lane)
                dstv = fill + (incl - mi)
                plsc.store_scatter(vbuf, [dstv], v, mask=m)
                plsc.store_scatter(rbuf, [dstv], rows, mask=m)
                plsc.store_scatter(cbuf, [dstv], cols, mask=m)
                cols2 = cols + L
                wrap = cols2 >= C
                return (fill + incl[15], rows + wrap.astype(jnp.int32),
                        jnp.where(wrap, cols2 - C, cols2))

            fill, _, _ = lax.fori_loop(0, TPW, body, (fill, rows0, lane))
            return flush(fill, flushed)

        fill, flushed = lax.fori_loop(
            0, NW, chunk_body, (jnp.int32(0), jnp.int32(0)))

        padlen = N - K
        hv = jnp.minimum(padlen, (8 - (fill & 7)) & 7)
        plsc.store_scatter(vbuf, [fill + lane], padv, mask=lane < hv)
        plsc.store_scatter(rbuf, [fill + lane], zerov, mask=lane < hv)
        plsc.store_scatter(cbuf, [fill + lane], zerov, mask=lane < hv)
        fill, flushed = flush(fill + hv, flushed)

        for q in range(64):
            sl = slice(q * L, (q + 1) * L)
            vbuf[sl] = padv
            rbuf[sl] = zerov
            cbuf[sl] = zerov

        rem = N - flushed

        def cbig(i, _):
            dst = _mult8(flushed + i * 1024)
            pltpu.sync_copy(vbuf.at[pl.ds(0, 1024)],
                            val_hbm.at[pl.ds(dst, 1024)])
            pltpu.sync_copy(rbuf.at[pl.ds(0, 1024)],
                            rows_hbm.at[pl.ds(dst, 1024)])
            pltpu.sync_copy(cbuf.at[pl.ds(0, 1024)],
                            cols_hbm.at[pl.ds(dst, 1024)])
            return 0

        lax.fori_loop(0, rem >> 10, cbig, 0)
        done = (rem >> 10) << 10

        def csm(i, _):
            dst = _mult8(flushed + done + i * 8)
            pltpu.sync_copy(vbuf.at[pl.ds(0, 8)], val_hbm.at[pl.ds(dst, 8)])
            pltpu.sync_copy(rbuf.at[pl.ds(0, 8)], rows_hbm.at[pl.ds(dst, 8)])
            pltpu.sync_copy(cbuf.at[pl.ds(0, 8)], cols_hbm.at[pl.ds(dst, 8)])
            return 0

        lax.fori_loop(0, (rem - done) >> 3, csm, 0)


def kernel(inputs):
    prefix = _prefix(inputs)
    values, rows, cols = _transform(inputs.reshape(N), prefix)
    indices = jnp.stack([rows, cols], axis=1)
    dense_shape = jnp.array([R, C], dtype=jnp.int32)
    return indices, values, dense_shape


# speedup vs baseline: 2.4872x; 2.4872x over previous
import functools

import jax
import jax.numpy as jnp
from jax import lax
from jax.experimental import pallas as pl
from jax.experimental.pallas import tpu as pltpu
from jax.experimental.pallas import tpu_sc as plsc

R, C = 4096, 200
N = R * C
L = 16
NC, NS = 2, 16
NW = NC * NS
CPW = N // NW
TPW = CPW // L
ROWS_PW = R // NW
IGNORE = -1.0


def _count_body(x_ref, out_ref):
    w = pl.program_id(0)

    @pl.when(w == 0)
    def _():
        out_ref[0] = 0

    c = jnp.sum((x_ref[...] != IGNORE).astype(jnp.int32))
    out_ref[w + 1] = out_ref[w] + c


def _prefix(x):
    return pl.pallas_call(
        _count_body,
        grid=(NW,),
        in_specs=[pl.BlockSpec((ROWS_PW, C), lambda i: (i, 0))],
        out_specs=pl.BlockSpec(memory_space=pltpu.SMEM),
        out_shape=jax.ShapeDtypeStruct((48,), jnp.int32),
    )(x)


def _gather16(x, idx):
    return x.at[idx].get(mode="promise_in_bounds")


def _cumsum16(x, lane):
    s = x
    for k in (1, 2, 4, 8):
        sh = _gather16(s, jnp.maximum(lane - k, 0))
        s = s + jnp.where(lane >= k, sh, 0)
    return s


def _mult8(x):
    return pl.multiple_of(x, 8)


SBUF = CPW + 32


@functools.partial(
    pl.kernel,
    mesh=plsc.VectorSubcoreMesh(core_axis_name="c", subcore_axis_name="s"),
    out_type=[
        jax.ShapeDtypeStruct((N,), jnp.float32),
        jax.ShapeDtypeStruct((N,), jnp.int32),
        jax.ShapeDtypeStruct((N,), jnp.int32),
    ],
    scratch_types=[
        pltpu.VMEM((CPW,), jnp.float32),
        pltpu.VMEM((SBUF,), jnp.float32),
        pltpu.VMEM((SBUF,), jnp.int32),
        pltpu.VMEM((SBUF,), jnp.int32),
        pltpu.VMEM((48,), jnp.int32),
        pltpu.SemaphoreType.DMA,
    ],
    compiler_params=pltpu.CompilerParams(
        needs_layout_passes=False, vmem_limit_bytes=65536),
)
def _transform(x_hbm, pref_hbm, val_hbm, rows_hbm, cols_hbm,
               chunk, vbuf, rbuf, cbuf, pref, sem):
    wid = lax.axis_index("s") * NC + lax.axis_index("c")
    base_elem = _mult8(wid * CPW)

    pltpu.sync_copy(pref_hbm, pref)
    lane = lax.iota(jnp.int32, L)
    K = pref[2 * L : 3 * L][0]

    @pl.when(K == N)
    def _fast():
        cin = pltpu.async_copy(x_hbm.at[pl.ds(base_elem, CPW)], chunk, sem)

        def body(t, carry):
            rows, cols = carry
            j = t * L
            rbuf[pl.ds(j, L)] = rows
            cbuf[pl.ds(j, L)] = cols
            cols2 = cols + L
            wrap = cols2 >= C
            return rows + wrap.astype(jnp.int32), jnp.where(wrap, cols2 - C, cols2)

        lax.fori_loop(0, TPW, body,
                      (jnp.broadcast_to(wid * ROWS_PW, (L,)), lane))
        cin.wait()
        pltpu.sync_copy(chunk, val_hbm.at[pl.ds(base_elem, CPW)])
        pltpu.sync_copy(rbuf.at[pl.ds(0, CPW)],
                        rows_hbm.at[pl.ds(base_elem, CPW)])
        pltpu.sync_copy(cbuf.at[pl.ds(0, CPW)],
                        cols_hbm.at[pl.ds(base_elem, CPW)])

    @pl.when(jnp.logical_and(K != N, wid == 0))
    def _slow():
        pltpu.sync_copy(x_hbm.at[pl.ds(0, L)], chunk.at[pl.ds(0, L)])
        padv = jnp.broadcast_to(chunk[0:L][0], (L,))
        zerov = jnp.zeros((L,), jnp.int32)

        def flush(fill, flushed):
            nbig = fill >> 10

            def fbig(i, _):
                src = _mult8(i * 1024)
                dst = _mult8(flushed + i * 1024)
                pltpu.sync_copy(vbuf.at[pl.ds(src, 1024)],
                                val_hbm.at[pl.ds(dst, 1024)])
                pltpu.sync_copy(rbuf.at[pl.ds(src, 1024)],
                                rows_hbm.at[pl.ds(dst, 1024)])
                pltpu.sync_copy(cbuf.at[pl.ds(src, 1024)],
                                cols_hbm.at[pl.ds(dst, 1024)])
                return 0

            lax.fori_loop(0, nbig, fbig, 0)
            base8 = nbig << 10
            nsm = (fill - base8) >> 3

            def fsm(i, _):
                src = _mult8(base8 + i * 8)
                dst = _mult8(flushed + base8 + i * 8)
                pltpu.sync_copy(vbuf.at[pl.ds(src, 8)],
                                val_hbm.at[pl.ds(dst, 8)])
                pltpu.sync_copy(rbuf.at[pl.ds(src, 8)],
                                rows_hbm.at[pl.ds(dst, 8)])
                pltpu.sync_copy(cbuf.at[pl.ds(src, 8)],
                                cols_hbm.at[pl.ds(dst, 8)])
                return 0

            lax.fori_loop(0, nsm, fsm, 0)
            done = _mult8((fill >> 3) << 3)
            tv = vbuf[pl.ds(done, L)]
            tr = rbuf[pl.ds(done, L)]
            tc = cbuf[pl.ds(done, L)]
            vbuf[0:L] = tv
            rbuf[0:L] = tr
            cbuf[0:L] = tc
            return fill - done, flushed + done

        def chunk_body(ch, carry):
            fill, flushed = carry
            pltpu.sync_copy(x_hbm.at[pl.ds(_mult8(ch * CPW), CPW)], chunk)
            rows0 = jnp.broadcast_to(ch * ROWS_PW, (L,))

            def body(t, c2):
                fill, rows, cols = c2
                v = chunk[pl.ds(t * L, L)]
                m = v != IGNORE
                mi = m.astype(jnp.int32)
                incl = _cumsum16(mi, lane)
                dstv = fill + (incl - mi)
                plsc.store_scatter(vbuf, [dstv], v, mask=m)
                plsc.store_scatter(rbuf, [dstv], rows, mask=m)
                plsc.store_scatter(cbuf, [dstv], cols, mask=m)
                cols2 = cols + L
                wrap = cols2 >= C
                return (fill + incl[15], rows + wrap.astype(jnp.int32),
                        jnp.where(wrap, cols2 - C, cols2))

            fill, _, _ = lax.fori_loop(0, TPW, body, (fill, rows0, lane))
            return flush(fill, flushed)

        fill, flushed = lax.fori_loop(
            0, NW, chunk_body, (jnp.int32(0), jnp.int32(0)))

        padlen = N - K
        hv = jnp.minimum(padlen, (8 - (fill & 7)) & 7)
        plsc.store_scatter(vbuf, [fill + lane], padv, mask=lane < hv)
        plsc.store_scatter(rbuf, [fill + lane], zerov, mask=lane < hv)
        plsc.store_scatter(cbuf, [fill + lane], zerov, mask=lane < hv)
        fill, flushed = flush(fill + hv, flushed)

        for q in range(64):
            sl = slice(q * L, (q + 1) * L)
            vbuf[sl] = padv
            rbuf[sl] = zerov
            cbuf[sl] = zerov

        rem = N - flushed

        def cbig(i, _):
            dst = _mult8(flushed + i * 1024)
            pltpu.sync_copy(vbuf.at[pl.ds(0, 1024)],
                            val_hbm.at[pl.ds(dst, 1024)])
            pltpu.sync_copy(rbuf.at[pl.ds(0, 1024)],
                            rows_hbm.at[pl.ds(dst, 1024)])
            pltpu.sync_copy(cbuf.at[pl.ds(0, 1024)],
                            cols_hbm.at[pl.ds(dst, 1024)])
            return 0

        lax.fori_loop(0, rem >> 10, cbig, 0)
        done = (rem >> 10) << 10

        def csm(i, _):
            dst = _mult8(flushed + done + i * 8)
            pltpu.sync_copy(vbuf.at[pl.ds(0, 8)], val_hbm.at[pl.ds(dst, 8)])
            pltpu.sync_copy(rbuf.at[pl.ds(0, 8)], rows_hbm.at[pl.ds(dst, 8)])
            pltpu.sync_copy(cbuf.at[pl.ds(0, 8)], cols_hbm.at[pl.ds(dst, 8)])
            return 0

        lax.fori_loop(0, (rem - done) >> 3, csm, 0)


def kernel(inputs):
    prefix = _prefix(inputs)
    values, rows, cols = _transform(inputs.reshape(N), prefix)
    indices = jnp.stack([rows, cols], axis=1)
    dense_shape = jnp.array([R, C], dtype=jnp.int32)
    return indices, values, dense_shape


# speedup vs baseline: 3.2000x; 1.2866x over previous
import functools

import jax
import jax.numpy as jnp
from jax import lax
from jax.experimental import pallas as pl
from jax.experimental.pallas import tpu as pltpu
from jax.experimental.pallas import tpu_sc as plsc

R, C = 4096, 200
N = R * C
L = 16
NC, NS = 2, 16
NW = NC * NS
CPW = N // NW
TPW = CPW // L
ROWS_PW = R // NW
IGNORE = -1.0


def _count_body(x_ref, out_ref):
    run = jnp.int32(0)
    out_ref[0] = run
    for w in range(NW):
        blk = x_ref[w * ROWS_PW : (w + 1) * ROWS_PW, :]
        run = run + jnp.sum((blk != IGNORE).astype(jnp.int32))
        out_ref[w + 1] = run


def _prefix(x):
    return pl.pallas_call(
        _count_body,
        out_specs=pl.BlockSpec(memory_space=pltpu.SMEM),
        out_shape=jax.ShapeDtypeStruct((48,), jnp.int32),
    )(x)


def _gather16(x, idx):
    return x.at[idx].get(mode="promise_in_bounds")


def _cumsum16(x, lane):
    s = x
    for k in (1, 2, 4, 8):
        sh = _gather16(s, jnp.maximum(lane - k, 0))
        s = s + jnp.where(lane >= k, sh, 0)
    return s


def _mult8(x):
    return pl.multiple_of(x, 8)


SBUF = CPW + 32


@functools.partial(
    pl.kernel,
    mesh=plsc.VectorSubcoreMesh(core_axis_name="c", subcore_axis_name="s"),
    out_type=[
        jax.ShapeDtypeStruct((N,), jnp.float32),
        jax.ShapeDtypeStruct((N,), jnp.int32),
        jax.ShapeDtypeStruct((N,), jnp.int32),
    ],
    scratch_types=[
        pltpu.VMEM((CPW,), jnp.float32),
        pltpu.VMEM((SBUF,), jnp.float32),
        pltpu.VMEM((SBUF,), jnp.int32),
        pltpu.VMEM((SBUF,), jnp.int32),
        pltpu.VMEM((48,), jnp.int32),
        pltpu.SemaphoreType.DMA,
    ],
    compiler_params=pltpu.CompilerParams(
        needs_layout_passes=False, vmem_limit_bytes=65536),
)
def _transform(x_hbm, pref_hbm, val_hbm, rows_hbm, cols_hbm,
               chunk, vbuf, rbuf, cbuf, pref, sem):
    wid = lax.axis_index("s") * NC + lax.axis_index("c")
    base_elem = _mult8(wid * CPW)

    pltpu.sync_copy(pref_hbm, pref)
    lane = lax.iota(jnp.int32, L)
    K = pref[2 * L : 3 * L][0]

    @pl.when(K == N)
    def _fast():
        cin = pltpu.async_copy(x_hbm.at[pl.ds(base_elem, CPW)], chunk, sem)

        def body(t, carry):
            rows, cols = carry
            j = t * L
            rbuf[pl.ds(j, L)] = rows
            cbuf[pl.ds(j, L)] = cols
            cols2 = cols + L
            wrap = cols2 >= C
            return rows + wrap.astype(jnp.int32), jnp.where(wrap, cols2 - C, cols2)

        lax.fori_loop(0, TPW, body,
                      (jnp.broadcast_to(wid * ROWS_PW, (L,)), lane))
        cin.wait()
        pltpu.sync_copy(chunk, val_hbm.at[pl.ds(base_elem, CPW)])
        pltpu.sync_copy(rbuf.at[pl.ds(0, CPW)],
                        rows_hbm.at[pl.ds(base_elem, CPW)])
        pltpu.sync_copy(cbuf.at[pl.ds(0, CPW)],
                        cols_hbm.at[pl.ds(base_elem, CPW)])

    @pl.when(jnp.logical_and(K != N, wid == 0))
    def _slow():
        pltpu.sync_copy(x_hbm.at[pl.ds(0, L)], chunk.at[pl.ds(0, L)])
        padv = jnp.broadcast_to(chunk[0:L][0], (L,))
        zerov = jnp.zeros((L,), jnp.int32)

        def flush(fill, flushed):
            nbig = fill >> 10

            def fbig(i, _):
                src = _mult8(i * 1024)
                dst = _mult8(flushed + i * 1024)
                pltpu.sync_copy(vbuf.at[pl.ds(src, 1024)],
                                val_hbm.at[pl.ds(dst, 1024)])
                pltpu.sync_copy(rbuf.at[pl.ds(src, 1024)],
                                rows_hbm.at[pl.ds(dst, 1024)])
                pltpu.sync_copy(cbuf.at[pl.ds(src, 1024)],
                                cols_hbm.at[pl.ds(dst, 1024)])
                return 0

            lax.fori_loop(0, nbig, fbig, 0)
            base8 = nbig << 10
            nsm = (fill - base8) >> 3

            def fsm(i, _):
                src = _mult8(base8 + i * 8)
                dst = _mult8(flushed + base8 + i * 8)
                pltpu.sync_copy(vbuf.at[pl.ds(src, 8)],
                                val_hbm.at[pl.ds(dst, 8)])
                pltpu.sync_copy(rbuf.at[pl.ds(src, 8)],
                                rows_hbm.at[pl.ds(dst, 8)])
                pltpu.sync_copy(cbuf.at[pl.ds(src, 8)],
                                cols_hbm.at[pl.ds(dst, 8)])
                return 0

            lax.fori_loop(0, nsm, fsm, 0)
            done = _mult8((fill >> 3) << 3)
            tv = vbuf[pl.ds(done, L)]
            tr = rbuf[pl.ds(done, L)]
            tc = cbuf[pl.ds(done, L)]
            vbuf[0:L] = tv
            rbuf[0:L] = tr
            cbuf[0:L] = tc
            return fill - done, flushed + done

        def chunk_body(ch, carry):
            fill, flushed = carry
            pltpu.sync_copy(x_hbm.at[pl.ds(_mult8(ch * CPW), CPW)], chunk)
            rows0 = jnp.broadcast_to(ch * ROWS_PW, (L,))

            def body(t, c2):
                fill, rows, cols = c2
                v = chunk[pl.ds(t * L, L)]
                m = v != IGNORE
                mi = m.astype(jnp.int32)
                incl = _cumsum16(mi, lane)
                dstv = fill + (incl - mi)
                plsc.store_scatter(vbuf, [dstv], v, mask=m)
                plsc.store_scatter(rbuf, [dstv], rows, mask=m)
                plsc.store_scatter(cbuf, [dstv], cols, mask=m)
                cols2 = cols + L
                wrap = cols2 >= C
                return (fill + incl[15], rows + wrap.astype(jnp.int32),
                        jnp.where(wrap, cols2 - C, cols2))

            fill, _, _ = lax.fori_loop(0, TPW, body, (fill, rows0, lane))
            return flush(fill, flushed)

        fill, flushed = lax.fori_loop(
            0, NW, chunk_body, (jnp.int32(0), jnp.int32(0)))

        padlen = N - K
        hv = jnp.minimum(padlen, (8 - (fill & 7)) & 7)
        plsc.store_scatter(vbuf, [fill + lane], padv, mask=lane < hv)
        plsc.store_scatter(rbuf, [fill + lane], zerov, mask=lane < hv)
        plsc.store_scatter(cbuf, [fill + lane], zerov, mask=lane < hv)
        fill, flushed = flush(fill + hv, flushed)

        for q in range(64):
            sl = slice(q * L, (q + 1) * L)
            vbuf[sl] = padv
            rbuf[sl] = zerov
            cbuf[sl] = zerov

        rem = N - flushed

        def cbig(i, _):
            dst = _mult8(flushed + i * 1024)
            pltpu.sync_copy(vbuf.at[pl.ds(0, 1024)],
                            val_hbm.at[pl.ds(dst, 1024)])
            pltpu.sync_copy(rbuf.at[pl.ds(0, 1024)],
                            rows_hbm.at[pl.ds(dst, 1024)])
            pltpu.sync_copy(cbuf.at[pl.ds(0, 1024)],
                            cols_hbm.at[pl.ds(dst, 1024)])
            return 0

        lax.fori_loop(0, rem >> 10, cbig, 0)
        done = (rem >> 10) << 10

        def csm(i, _):
            dst = _mult8(flushed + done + i * 8)
            pltpu.sync_copy(vbuf.at[pl.ds(0, 8)], val_hbm.at[pl.ds(dst, 8)])
            pltpu.sync_copy(rbuf.at[pl.ds(0, 8)], rows_hbm.at[pl.ds(dst, 8)])
            pltpu.sync_copy(cbuf.at[pl.ds(0, 8)], cols_hbm.at[pl.ds(dst, 8)])
            return 0

        lax.fori_loop(0, (rem - done) >> 3, csm, 0)


def kernel(inputs):
    prefix = _prefix(inputs)
    values, rows, cols = _transform(inputs.reshape(N), prefix)
    indices = jnp.stack([rows, cols], axis=1)
    dense_shape = jnp.array([R, C], dtype=jnp.int32)
    return indices, values, dense_shape


# speedup vs baseline: 3.2083x; 1.0026x over previous
import functools

import jax
import jax.numpy as jnp
from jax import lax
from jax.experimental import pallas as pl
from jax.experimental.pallas import tpu as pltpu
from jax.experimental.pallas import tpu_sc as plsc

R, C = 4096, 200
N = R * C
L = 16
NC, NS = 2, 16
NW = NC * NS
CPW = N // NW
TPW = CPW // L
ROWS_PW = R // NW
IGNORE = -1.0


def _count_body(x_ref, out_ref):
    run = jnp.int32(0)
    out_ref[0] = run
    for w in range(NW):
        blk = x_ref[w * ROWS_PW : (w + 1) * ROWS_PW, :]
        run = run + jnp.sum((blk != IGNORE).astype(jnp.int32))
        out_ref[w + 1] = run


def _prefix(x):
    return pl.pallas_call(
        _count_body,
        out_specs=pl.BlockSpec(memory_space=pltpu.SMEM),
        out_shape=jax.ShapeDtypeStruct((48,), jnp.int32),
    )(x)


def _gather16(x, idx):
    return x.at[idx].get(mode="promise_in_bounds")


def _cumsum16(x, lane):
    s = x
    for k in (1, 2, 4, 8):
        sh = _gather16(s, jnp.maximum(lane - k, 0))
        s = s + jnp.where(lane >= k, sh, 0)
    return s


def _mult8(x):
    return pl.multiple_of(x, 8)


SBUF = CPW + 32


@functools.partial(
    pl.kernel,
    mesh=plsc.VectorSubcoreMesh(core_axis_name="c", subcore_axis_name="s"),
    out_type=[
        jax.ShapeDtypeStruct((N,), jnp.float32),
        jax.ShapeDtypeStruct((N,), jnp.int32),
        jax.ShapeDtypeStruct((N,), jnp.int32),
    ],
    scratch_types=[
        pltpu.VMEM((CPW,), jnp.float32),
        pltpu.VMEM((SBUF,), jnp.float32),
        pltpu.VMEM((SBUF,), jnp.int32),
        pltpu.VMEM((SBUF,), jnp.int32),
        pltpu.VMEM((48,), jnp.int32),
        pltpu.SemaphoreType.DMA,
    ],
    compiler_params=pltpu.CompilerParams(
        needs_layout_passes=False, vmem_limit_bytes=65536),
)
def _transform(x_hbm, pref_hbm, val_hbm, rows_hbm, cols_hbm,
               chunk, vbuf, rbuf, cbuf, pref, sem):
    wid = lax.axis_index("s") * NC + lax.axis_index("c")
    base_elem = _mult8(wid * CPW)

    pltpu.sync_copy(pref_hbm, pref)
    lane = lax.iota(jnp.int32, L)
    K = pref[2 * L : 3 * L][0]

    @pl.when(K == N)
    def _fast():
        cin = pltpu.async_copy(x_hbm.at[pl.ds(base_elem, CPW)], chunk, sem)

        def body(t, carry):
            rows, cols = carry
            j = t * L
            rbuf[pl.ds(j, L)] = rows
            cbuf[pl.ds(j, L)] = cols
            cols2 = cols + L
            wrap = cols2 >= C
            return rows + wrap.astype(jnp.int32), jnp.where(wrap, cols2 - C, cols2)

        lax.fori_loop(0, TPW, body,
                      (jnp.broadcast_to(wid * ROWS_PW, (L,)), lane))
        cin.wait()
        c1 = pltpu.async_copy(chunk, val_hbm.at[pl.ds(base_elem, CPW)], sem)
        c2 = pltpu.async_copy(rbuf.at[pl.ds(0, CPW)],
                              rows_hbm.at[pl.ds(base_elem, CPW)], sem)
        c3 = pltpu.async_copy(cbuf.at[pl.ds(0, CPW)],
                              cols_hbm.at[pl.ds(base_elem, CPW)], sem)
        c1.wait()
        c2.wait()
        c3.wait()

    @pl.when(jnp.logical_and(K != N, wid == 0))
    def _slow():
        pltpu.sync_copy(x_hbm.at[pl.ds(0, L)], chunk.at[pl.ds(0, L)])
        padv = jnp.broadcast_to(chunk[0:L][0], (L,))
        zerov = jnp.zeros((L,), jnp.int32)

        def flush(fill, flushed):
            nbig = fill >> 10

            def fbig(i, _):
                src = _mult8(i * 1024)
                dst = _mult8(flushed + i * 1024)
                pltpu.sync_copy(vbuf.at[pl.ds(src, 1024)],
                                val_hbm.at[pl.ds(dst, 1024)])
                pltpu.sync_copy(rbuf.at[pl.ds(src, 1024)],
                                rows_hbm.at[pl.ds(dst, 1024)])
                pltpu.sync_copy(cbuf.at[pl.ds(src, 1024)],
                                cols_hbm.at[pl.ds(dst, 1024)])
                return 0

            lax.fori_loop(0, nbig, fbig, 0)
            base8 = nbig << 10
            nsm = (fill - base8) >> 3

            def fsm(i, _):
                src = _mult8(base8 + i * 8)
                dst = _mult8(flushed + base8 + i * 8)
                pltpu.sync_copy(vbuf.at[pl.ds(src, 8)],
                                val_hbm.at[pl.ds(dst, 8)])
                pltpu.sync_copy(rbuf.at[pl.ds(src, 8)],
                                rows_hbm.at[pl.ds(dst, 8)])
                pltpu.sync_copy(cbuf.at[pl.ds(src, 8)],
                                cols_hbm.at[pl.ds(dst, 8)])
                return 0

            lax.fori_loop(0, nsm, fsm, 0)
            done = _mult8((fill >> 3) << 3)
            tv = vbuf[pl.ds(done, L)]
            tr = rbuf[pl.ds(done, L)]
            tc = cbuf[pl.ds(done, L)]
            vbuf[0:L] = tv
            rbuf[0:L] = tr
            cbuf[0:L] = tc
            return fill - done, flushed + done

        def chunk_body(ch, carry):
            fill, flushed = carry
            pltpu.sync_copy(x_hbm.at[pl.ds(_mult8(ch * CPW), CPW)], chunk)
            rows0 = jnp.broadcast_to(ch * ROWS_PW, (L,))

            def body(t, c2):
                fill, rows, cols = c2
                v = chunk[pl.ds(t * L, L)]
                m = v != IGNORE
                mi = m.astype(jnp.int32)
                incl = _cumsum16(mi, lane)
                dstv = fill + (incl - mi)
                plsc.store_scatter(vbuf, [dstv], v, mask=m)
                plsc.store_scatter(rbuf, [dstv], rows, mask=m)
                plsc.store_scatter(cbuf, [dstv], cols, mask=m)
                cols2 = cols + L
                wrap = cols2 >= C
                return (fill + incl[15], rows + wrap.astype(jnp.int32),
                        jnp.where(wrap, cols2 - C, cols2))

            fill, _, _ = lax.fori_loop(0, TPW, body, (fill, rows0, lane))
            return flush(fill, flushed)

        fill, flushed = lax.fori_loop(
            0, NW, chunk_body, (jnp.int32(0), jnp.int32(0)))

        padlen = N - K
        hv = jnp.minimum(padlen, (8 - (fill & 7)) & 7)
        plsc.store_scatter(vbuf, [fill + lane], padv, mask=lane < hv)
        plsc.store_scatter(rbuf, [fill + lane], zerov, mask=lane < hv)
        plsc.store_scatter(cbuf, [fill + lane], zerov, mask=lane < hv)
        fill, flushed = flush(fill + hv, flushed)

        for q in range(64):
            sl = slice(q * L, (q + 1) * L)
            vbuf[sl] = padv
            rbuf[sl] = zerov
            cbuf[sl] = zerov

        rem = N - flushed

        def cbig(i, _):
            dst = _mult8(flushed + i * 1024)
            pltpu.sync_copy(vbuf.at[pl.ds(0, 1024)],
                            val_hbm.at[pl.ds(dst, 1024)])
            pltpu.sync_copy(rbuf.at[pl.ds(0, 1024)],
                            rows_hbm.at[pl.ds(dst, 1024)])
            pltpu.sync_copy(cbuf.at[pl.ds(0, 1024)],
                            cols_hbm.at[pl.ds(dst, 1024)])
            return 0

        lax.fori_loop(0, rem >> 10, cbig, 0)
        done = (rem >> 10) << 10

        def csm(i, _):
            dst = _mult8(flushed + done + i * 8)
            pltpu.sync_copy(vbuf.at[pl.ds(0, 8)], val_hbm.at[pl.ds(dst, 8)])
            pltpu.sync_copy(rbuf.at[pl.ds(0, 8)], rows_hbm.at[pl.ds(dst, 8)])
            pltpu.sync_copy(cbuf.at[pl.ds(0, 8)], cols_hbm.at[pl.ds(dst, 8)])
            return 0

        lax.fori_loop(0, (rem - done) >> 3, csm, 0)


def kernel(inputs):
    prefix = _prefix(inputs)
    values, rows, cols = _transform(inputs.reshape(N), prefix)
    indices = jnp.stack([rows, cols], axis=1)
    dense_shape = jnp.array([R, C], dtype=jnp.int32)
    return indices, values, dense_shape
